# baseline (speedup 1.0000x reference)
import jax
import jax.numpy as jnp
from jax.experimental import pallas as pl
from jax.experimental.pallas import tpu as pltpu

_RADIUS = 2
_WIN = 2 * _RADIUS + 1
_TILE_H = 28


def _body(idx_ref, fields_ref, out_ref, win_ref, mean_ref, sem):
    b = pl.program_id(0)
    t = pl.program_id(1)

    @pl.when(t == 0)
    def _():
        py = idx_ref[b, 0]
        px = idx_ref[b, 1]
        cp = pltpu.make_async_copy(
            fields_ref.at[b, pl.ds(py - _RADIUS, _WIN), pl.ds(px - _RADIUS, _WIN), :],
            win_ref,
            sem,
        )
        cp.start()
        cp.wait()
        w = win_ref[:, :, 2:]
        mean_ref[0, :] = jnp.sum(w, axis=(0, 1)) * (1.0 / (_WIN * _WIN))

    out_ref[...] = jnp.broadcast_to(
        mean_ref[0, :][None, None, None, :], out_ref.shape
    )


def kernel(fields, pump_indices):
    B, H, W, C = fields.shape
    Cout = C - 2
    idx = pump_indices.astype(jnp.int32)

    grid_spec = pltpu.PrefetchScalarGridSpec(
        num_scalar_prefetch=1,
        grid=(B, H // _TILE_H),
        in_specs=[
            pl.BlockSpec(memory_space=pl.ANY),
        ],
        out_specs=pl.BlockSpec(
            (1, _TILE_H, W, Cout), lambda b, t, idx_ref: (b, t, 0, 0)
        ),
        scratch_shapes=[
            pltpu.VMEM((_WIN, _WIN, C), jnp.float32),
            pltpu.VMEM((1, Cout), jnp.float32),
            pltpu.SemaphoreType.DMA,
        ],
    )

    return pl.pallas_call(
        _body,
        grid_spec=grid_spec,
        out_shape=jax.ShapeDtypeStruct((B, H, W, Cout), jnp.float32),
    )(idx, fields)
